# baseline (speedup 1.0000x reference)
import functools

import jax
import jax.numpy as jnp
from jax.experimental import pallas as pl
from jax.experimental.pallas import tpu as pltpu

EMBED_DIM = 1024
NUM_EXPERTS = 8
TOP_K = 2
N_TOKENS = 4096

T_GATE = 512
T_FFN = 512

NEG_INF = float("-inf")


def _gating_body(x_ref, wg_ref, logits_ref, idx_ref, we_ref):
    x = x_ref[...]
    wg = wg_ref[...]
    logits = jnp.dot(x, wg, preferred_element_type=jnp.float32)
    logits_ref[...] = logits

    e_iota = jax.lax.broadcasted_iota(jnp.int32, logits.shape, 1)
    m1 = jnp.max(logits, axis=1, keepdims=True)
    idx1 = jnp.min(jnp.where(logits == m1, e_iota, NUM_EXPERTS), axis=1,
                   keepdims=True)
    masked = jnp.where(e_iota == idx1, NEG_INF, logits)
    m2 = jnp.max(masked, axis=1, keepdims=True)
    idx2 = jnp.min(jnp.where(masked == m2, e_iota, NUM_EXPERTS), axis=1,
                   keepdims=True)
    s = jnp.exp(m2 - m1)
    p1 = 1.0 / (1.0 + s)
    p2 = s / (1.0 + s)
    idx_ref[...] = jnp.concatenate([idx1, idx2], axis=1)
    we_ref[...] = jnp.where(e_iota == idx1, p1, 0.0) + jnp.where(
        e_iota == idx2, p2, 0.0)


def _gating(x, Wg):
    n_tiles = N_TOKENS // T_GATE
    out_shapes = (
        jax.ShapeDtypeStruct((N_TOKENS, NUM_EXPERTS), jnp.float32),
        jax.ShapeDtypeStruct((N_TOKENS, TOP_K), jnp.int32),
        jax.ShapeDtypeStruct((N_TOKENS, NUM_EXPERTS), jnp.float32),
    )
    return pl.pallas_call(
        _gating_body,
        grid=(n_tiles,),
        in_specs=[
            pl.BlockSpec((T_GATE, EMBED_DIM), lambda t: (t, 0)),
            pl.BlockSpec((EMBED_DIM, NUM_EXPERTS), lambda t: (0, 0)),
        ],
        out_specs=(
            pl.BlockSpec((T_GATE, NUM_EXPERTS), lambda t: (t, 0)),
            pl.BlockSpec((T_GATE, TOP_K), lambda t: (t, 0)),
            pl.BlockSpec((T_GATE, NUM_EXPERTS), lambda t: (t, 0)),
        ),
        out_shape=out_shapes,
    )(x, Wg)


def _ffn_dense_body(x_ref, w1_ref, b1_ref, w2_ref, b2_ref, we_ref, out_ref):
    e = pl.program_id(1)
    x = x_ref[...]
    h = jnp.maximum(
        jnp.dot(x, w1_ref[0], preferred_element_type=jnp.float32) + b1_ref[0, 0],
        0.0)
    y = jnp.dot(h, w2_ref[0], preferred_element_type=jnp.float32) + b2_ref[0, 0]
    y = y * we_ref[0]

    @pl.when(e == 0)
    def _():
        out_ref[...] = y

    @pl.when(e != 0)
    def _():
        out_ref[...] = out_ref[...] + y


def _ffn_dense(x, W1, b1, W2, b2, we):
    n_tiles = N_TOKENS // T_FFN
    we3 = we.T.reshape(NUM_EXPERTS, N_TOKENS, 1)
    b1r = b1.reshape(NUM_EXPERTS, 1, 2 * EMBED_DIM)
    b2r = b2.reshape(NUM_EXPERTS, 1, EMBED_DIM)
    return pl.pallas_call(
        _ffn_dense_body,
        grid=(n_tiles, NUM_EXPERTS),
        in_specs=[
            pl.BlockSpec((T_FFN, EMBED_DIM), lambda t, e: (t, 0)),
            pl.BlockSpec((1, EMBED_DIM, 2 * EMBED_DIM), lambda t, e: (e, 0, 0)),
            pl.BlockSpec((1, 1, 2 * EMBED_DIM), lambda t, e: (e, 0, 0)),
            pl.BlockSpec((1, 2 * EMBED_DIM, EMBED_DIM), lambda t, e: (e, 0, 0)),
            pl.BlockSpec((1, 1, EMBED_DIM), lambda t, e: (e, 0, 0)),
            pl.BlockSpec((1, T_FFN, 1), lambda t, e: (e, t, 0)),
        ],
        out_specs=pl.BlockSpec((T_FFN, EMBED_DIM), lambda t, e: (t, 0)),
        out_shape=jax.ShapeDtypeStruct((N_TOKENS, EMBED_DIM), jnp.float32),
    )(x, W1, b1r, W2, b2r, we3)


@jax.jit
def kernel(x, Wg, W1, b1, W2, b2):
    clean_logits, top_idx, we = _gating(x, Wg)
    moe_output = _ffn_dense(x, W1, b1, W2, b2, we)
    return (moe_output, clean_logits, top_idx)


# speedup vs baseline: 1.6362x; 1.6362x over previous
import functools

import jax
import jax.numpy as jnp
from jax import lax
from jax.experimental import pallas as pl
from jax.experimental.pallas import tpu as pltpu
from jax.experimental.pallas import tpu_sc as plsc

D = 1024
H = 2048
E = 8
K = 2
N = 4096

T_GATE = 512
T_BLK = 256
NB = (N * K) // T_BLK + E
P = NB * T_BLK

NC = 2
NS = 16
NW = NC * NS
TPW = N // NW
CH = 64

NEG_INF = float("-inf")


def _gating_body(x_ref, wg_ref, logits_ref, idx_ref, probs_ref, rank_ref,
                 counts_ref, bases_ref):
    t = pl.program_id(0)
    x = x_ref[...]
    logits = jnp.dot(x, wg_ref[...], preferred_element_type=jnp.float32)
    logits_ref[...] = logits

    e_iota = lax.broadcasted_iota(jnp.int32, logits.shape, 1)
    m1 = jnp.max(logits, axis=1, keepdims=True)
    idx1 = jnp.min(jnp.where(logits == m1, e_iota, E), axis=1, keepdims=True)
    masked = jnp.where(e_iota == idx1, NEG_INF, logits)
    m2 = jnp.max(masked, axis=1, keepdims=True)
    idx2 = jnp.min(jnp.where(masked == m2, e_iota, E), axis=1, keepdims=True)
    s = jnp.exp(m2 - m1)
    p1 = 1.0 / (1.0 + s)
    p2 = s / (1.0 + s)
    idx_ref[...] = jnp.concatenate([idx1, idx2], axis=1)
    probs_ref[...] = jnp.concatenate([p1, p2], axis=1)

    oh1 = (e_iota == idx1).astype(jnp.float32)
    oh2 = (e_iota == idx2).astype(jnp.float32)
    cnt = oh1 + oh2
    r_io = lax.broadcasted_iota(jnp.int32, (T_GATE, T_GATE), 0)
    c_io = lax.broadcasted_iota(jnp.int32, (T_GATE, T_GATE), 1)
    tri = (r_io > c_io).astype(jnp.float32)
    c_excl = jnp.dot(tri, cnt, preferred_element_type=jnp.float32)

    @pl.when(t == 0)
    def _():
        bases_ref[...] = jnp.zeros((1, E), jnp.float32)

    g = bases_ref[...] + c_excl
    rank1 = jnp.sum(oh1 * g, axis=1, keepdims=True)
    rank2 = jnp.sum(oh2 * g, axis=1, keepdims=True)
    rank_ref[...] = jnp.concatenate([rank1, rank2], axis=1).astype(jnp.int32)

    new_bases = bases_ref[...] + jnp.sum(cnt, axis=0, keepdims=True)
    bases_ref[...] = new_bases
    counts_ref[...] = new_bases.astype(jnp.int32)


def _gating(x, Wg):
    n_tiles = N // T_GATE
    out_shapes = (
        jax.ShapeDtypeStruct((N, E), jnp.float32),
        jax.ShapeDtypeStruct((N, K), jnp.int32),
        jax.ShapeDtypeStruct((N, K), jnp.float32),
        jax.ShapeDtypeStruct((N, K), jnp.int32),
        jax.ShapeDtypeStruct((1, E), jnp.int32),
    )
    return pl.pallas_call(
        _gating_body,
        grid=(n_tiles,),
        in_specs=[
            pl.BlockSpec((T_GATE, D), lambda t: (t, 0)),
            pl.BlockSpec((D, E), lambda t: (0, 0)),
        ],
        out_specs=(
            pl.BlockSpec((T_GATE, E), lambda t: (t, 0)),
            pl.BlockSpec((T_GATE, K), lambda t: (t, 0)),
            pl.BlockSpec((T_GATE, K), lambda t: (t, 0)),
            pl.BlockSpec((T_GATE, K), lambda t: (t, 0)),
            pl.BlockSpec((1, E), lambda t: (0, 0)),
        ),
        out_shape=out_shapes,
        scratch_shapes=[pltpu.VMEM((1, E), jnp.float32)],
    )(x, Wg)


def _dispatch_sc(x, d0, d1):
    mesh = plsc.VectorSubcoreMesh(core_axis_name="c", subcore_axis_name="s")

    @functools.partial(
        pl.kernel,
        mesh=mesh,
        out_type=jax.ShapeDtypeStruct((P, D), jnp.float32),
        scratch_types=[
            pltpu.VMEM((CH, D), jnp.float32),
            pltpu.VMEM((CH,), jnp.int32),
            pltpu.VMEM((CH,), jnp.int32),
            pltpu.SemaphoreType.DMA,
        ],
    )
    def k(x_hbm, d0_hbm, d1_hbm, xg_hbm, xbuf, i0, i1, sem):
        wid = lax.axis_index("s") * NC + lax.axis_index("c")
        for c in range(TPW // CH):
            base = wid * TPW + c * CH
            pltpu.sync_copy(d0_hbm.at[pl.ds(base, CH)], i0)
            pltpu.sync_copy(d1_hbm.at[pl.ds(base, CH)], i1)
            pltpu.sync_copy(x_hbm.at[pl.ds(base, CH)], xbuf)
            pltpu.async_copy(xbuf, xg_hbm.at[i0], sem).wait()
            pltpu.async_copy(xbuf, xg_hbm.at[i1], sem).wait()

    return k(x, d0, d1)


def _ffn_body(be_ref, xg_ref, w1_ref, b1_ref, w2_ref, b2_ref, yb_ref):
    h = jnp.maximum(
        jnp.dot(xg_ref[...], w1_ref[0], preferred_element_type=jnp.float32)
        + b1_ref[0, 0], 0.0)
    yb_ref[...] = (
        jnp.dot(h, w2_ref[0], preferred_element_type=jnp.float32) + b2_ref[0, 0])


def _ffn(block_expert, xg, W1, b1r, W2, b2r):
    grid_spec = pltpu.PrefetchScalarGridSpec(
        num_scalar_prefetch=1,
        grid=(NB,),
        in_specs=[
            pl.BlockSpec((T_BLK, D), lambda b, be: (b, 0)),
            pl.BlockSpec((1, D, H), lambda b, be: (be[b], 0, 0)),
            pl.BlockSpec((1, 1, H), lambda b, be: (be[b], 0, 0)),
            pl.BlockSpec((1, H, D), lambda b, be: (be[b], 0, 0)),
            pl.BlockSpec((1, 1, D), lambda b, be: (be[b], 0, 0)),
        ],
        out_specs=pl.BlockSpec((T_BLK, D), lambda b, be: (b, 0)),
    )
    return pl.pallas_call(
        _ffn_body,
        grid_spec=grid_spec,
        out_shape=jax.ShapeDtypeStruct((P, D), jnp.float32),
    )(block_expert, xg, W1, b1r, W2, b2r)


def _combine_sc(yb, d0, d1):
    mesh = plsc.VectorSubcoreMesh(core_axis_name="c", subcore_axis_name="s")

    @functools.partial(
        pl.kernel,
        mesh=mesh,
        out_type=(
            jax.ShapeDtypeStruct((N, D), jnp.float32),
            jax.ShapeDtypeStruct((N, D), jnp.float32),
        ),
        scratch_types=[
            pltpu.VMEM((CH, D), jnp.float32),
            pltpu.VMEM((CH,), jnp.int32),
            pltpu.SemaphoreType.DMA,
        ],
    )
    def k(yb_hbm, d0_hbm, d1_hbm, y0_hbm, y1_hbm, buf, idx, sem):
        wid = lax.axis_index("s") * NC + lax.axis_index("c")
        for c in range(TPW // CH):
            base = wid * TPW + c * CH
            pltpu.sync_copy(d0_hbm.at[pl.ds(base, CH)], idx)
            pltpu.async_copy(yb_hbm.at[idx], buf, sem).wait()
            pltpu.sync_copy(buf, y0_hbm.at[pl.ds(base, CH)])
            pltpu.sync_copy(d1_hbm.at[pl.ds(base, CH)], idx)
            pltpu.async_copy(yb_hbm.at[idx], buf, sem).wait()
            pltpu.sync_copy(buf, y1_hbm.at[pl.ds(base, CH)])

    return k(yb, d0, d1)


def _combine_add_body(y0_ref, y1_ref, p0_ref, p1_ref, out_ref):
    out_ref[...] = p0_ref[...] * y0_ref[...] + p1_ref[...] * y1_ref[...]


def _combine_add(y0, y1, p0, p1):
    n_tiles = N // T_GATE
    return pl.pallas_call(
        _combine_add_body,
        grid=(n_tiles,),
        in_specs=[
            pl.BlockSpec((T_GATE, D), lambda t: (t, 0)),
            pl.BlockSpec((T_GATE, D), lambda t: (t, 0)),
            pl.BlockSpec((T_GATE, 1), lambda t: (t, 0)),
            pl.BlockSpec((T_GATE, 1), lambda t: (t, 0)),
        ],
        out_specs=pl.BlockSpec((T_GATE, D), lambda t: (t, 0)),
        out_shape=jax.ShapeDtypeStruct((N, D), jnp.float32),
    )(y0, y1, p0, p1)


@jax.jit
def kernel(x, Wg, W1, b1, W2, b2):
    clean_logits, top_idx, probs, rank, counts = _gating(x, Wg)

    counts = counts[0]
    padded = ((counts + T_BLK - 1) // T_BLK) * T_BLK
    pend = jnp.cumsum(padded)
    pstart = pend - padded
    e_ar = jnp.arange(E, dtype=jnp.int32)
    start_per_assign = jnp.sum(
        jnp.where(top_idx[:, :, None] == e_ar[None, None, :],
                  pstart[None, None, :], 0), axis=2)
    dest = rank + start_per_assign
    d0 = dest[:, 0]
    d1 = dest[:, 1]
    block_expert = jnp.minimum(
        jnp.sum((jnp.arange(NB, dtype=jnp.int32)[:, None] * T_BLK
                 >= pend[None, :]).astype(jnp.int32), axis=1),
        E - 1).astype(jnp.int32)

    xg = _dispatch_sc(x, d0, d1)

    b1r = b1.reshape(E, 1, H)
    b2r = b2.reshape(E, 1, D)
    yb = _ffn(block_expert, xg, W1, b1r, W2, b2r)

    y0, y1 = _combine_sc(yb, d0, d1)
    moe_output = _combine_add(y0, y1, probs[:, 0:1], probs[:, 1:2])
    return (moe_output, clean_logits, top_idx)
